# baseline (speedup 1.0000x reference)
import functools

import jax
import jax.numpy as jnp
from jax.experimental import pallas as pl
from jax.experimental.pallas import tpu as pltpu

_N = 4096
_SIGMA = 0.5
_BM = 512
_BS = 1024

_f32 = jnp.float32
_bf16 = jnp.bfloat16


def _dot(a, b):
    return jnp.dot(a.astype(_bf16), b.astype(_bf16),
                   preferred_element_type=_f32)


def _k1_body(x_ref, we1, be1, we2, be2, we3, be3,
             wd1, bd1, wd2, bd2, wd3, bd3, wxb, bxb, wg1,
             z1_ref, z2t_ref, z3_ref, z3t_ref, y1_ref, xhat_ref):
    x = x_ref[...]
    z1 = jax.nn.relu(_dot(x, we1[...]) + be1[...])
    z2 = jax.nn.relu(_dot(z1, we2[...]) + be2[...])
    z3 = _dot(z2, we3[...]) + be3[...]
    z1_ref[...] = z1
    z2t_ref[...] = z2.T
    z3_ref[...] = z3
    z3t_ref[...] = z3.T
    y1_ref[...] = _dot(x, wg1[...])
    d = jax.nn.relu(_dot(z3, wd1[...]) + bd1[...])
    d = jax.nn.relu(_dot(d, wd2[...]) + bd2[...])
    d = jax.nn.relu(_dot(d, wd3[...]) + bd3[...])
    xhat_ref[...] = _dot(d, wxb[...]) + bxb[...]


def _full(a):
    nd = a.ndim
    return pl.BlockSpec(a.shape, lambda i: (0,) * nd)


def _ae_chain(x, we1, be1, we2, be2, we3, be3,
              wd1, bd1, wd2, bd2, wd3, bd3, wxb, bxb, wg1_bf):
    n = x.shape[0]
    grid = (n // _BM,)
    row = lambda k: pl.BlockSpec((_BM, k), lambda i: (i, 0))
    consts = [we1, be1, we2, be2, we3, be3,
              wd1, bd1, wd2, bd2, wd3, bd3, wxb, bxb, wg1_bf]
    return pl.pallas_call(
        _k1_body,
        grid=grid,
        in_specs=[row(512)] + [_full(c) for c in consts],
        out_specs=[row(128),
                   pl.BlockSpec((64, _BM), lambda i: (0, i)),
                   row(20),
                   pl.BlockSpec((20, _BM), lambda i: (0, i)),
                   row(128), row(512)],
        out_shape=[
            jax.ShapeDtypeStruct((n, 128), _f32),
            jax.ShapeDtypeStruct((64, n), _f32),
            jax.ShapeDtypeStruct((n, 20), _f32),
            jax.ShapeDtypeStruct((20, n), _f32),
            jax.ShapeDtypeStruct((n, 128), _f32),
            jax.ShapeDtypeStruct((n, 512), _f32),
        ],
    )(x, *consts)


def _spmm_body(flags, *refs):
    (act, has_mix, has_next, want_t32, want_tbf, want_u32, want_ubf,
     cast_out) = flags
    it = iter(refs)
    adj_ref = next(it)
    y_ref = next(it)
    mix_ref = next(it) if has_mix else None
    w_ref = next(it) if has_next else None
    adj_blk = adj_ref[...].astype(_bf16) if cast_out else adj_ref[...]
    acc = jnp.dot(adj_blk, y_ref[...].astype(_bf16),
                  preferred_element_type=_f32)
    t = jnp.tanh(acc) if act else acc
    u = (1.0 - _SIGMA) * mix_ref[...] + _SIGMA * t if has_mix else t
    if want_t32:
        next(it)[...] = t
    if want_tbf:
        next(it)[...] = t.astype(_bf16)
    if want_u32:
        next(it)[...] = u
    if want_ubf:
        next(it)[...] = u.astype(_bf16)
    if has_next:
        next(it)[...] = _dot(u, w_ref[...])
    if cast_out:
        next(it)[...] = adj_blk


def _spmm_stage(adj_bf, y, mix=None, w_next=None, act=True,
                want_t32=False, want_tbf=False, want_u32=False,
                want_ubf=False, cast_out=False):
    n = adj_bf.shape[0]
    ncol = adj_bf.shape[1]
    kin = y.shape[1]
    grid = (n // _BM,)
    row = lambda k: pl.BlockSpec((_BM, k), lambda i: (i, 0))
    in_specs = [pl.BlockSpec((_BM, ncol), lambda i: (i, 0)), _full(y)]
    operands = [adj_bf, y]
    if mix is not None:
        in_specs.append(row(kin))
        operands.append(mix)
    if w_next is not None:
        in_specs.append(_full(w_next))
        operands.append(w_next)
    out_specs, out_shape = [], []
    if want_t32:
        out_specs.append(row(kin))
        out_shape.append(jax.ShapeDtypeStruct((n, kin), _f32))
    if want_tbf:
        out_specs.append(row(kin))
        out_shape.append(jax.ShapeDtypeStruct((n, kin), _bf16))
    if want_u32:
        out_specs.append(row(kin))
        out_shape.append(jax.ShapeDtypeStruct((n, kin), _f32))
    if want_ubf:
        out_specs.append(row(kin))
        out_shape.append(jax.ShapeDtypeStruct((n, kin), _bf16))
    if w_next is not None:
        kout = w_next.shape[1]
        out_specs.append(row(kout))
        out_shape.append(jax.ShapeDtypeStruct((n, kout), _f32))
    if cast_out:
        out_specs.append(pl.BlockSpec((_BM, ncol), lambda i: (i, 0)))
        out_shape.append(jax.ShapeDtypeStruct((n, ncol), _bf16))
    flags = (act, mix is not None, w_next is not None,
             want_t32, want_tbf, want_u32, want_ubf, cast_out)
    outs = pl.pallas_call(
        functools.partial(_spmm_body, flags),
        grid=grid,
        in_specs=in_specs,
        out_specs=out_specs,
        out_shape=out_shape,
    )(*operands)
    return outs


def _spmmT_body(flags, *refs):
    (act, has_mix, has_next, want_t32, want_u32, want_ubf) = flags
    it = iter(refs)
    adj_ref = next(it)
    y_ref = next(it)
    mixT_ref = next(it) if has_mix else None
    w_ref = next(it) if has_next else None
    tT = jax.lax.dot_general(y_ref[...].astype(_bf16), adj_ref[...],
                             (((0,), (1,)), ((), ())),
                             preferred_element_type=_f32)
    if act:
        tT = jnp.tanh(tT)
    uT = (1.0 - _SIGMA) * mixT_ref[...] + _SIGMA * tT if has_mix else tT
    if want_t32:
        next(it)[...] = tT.T
    if want_u32:
        next(it)[...] = uT.T
    if want_ubf:
        next(it)[...] = uT.T.astype(_bf16)
    if has_next:
        next(it)[...] = jax.lax.dot_general(uT.astype(_bf16),
                                            w_ref[...].astype(_bf16),
                                            (((0,), (0,)), ((), ())),
                                            preferred_element_type=_f32)


def _spmmT_stage(adj_bf, y, mixT=None, w_next=None, act=True,
                 want_t32=False, want_u32=False, want_ubf=False):
    n = adj_bf.shape[0]
    ncol = adj_bf.shape[1]
    kin = y.shape[1]
    grid = (n // _BM,)
    row = lambda k: pl.BlockSpec((_BM, k), lambda i: (i, 0))
    in_specs = [pl.BlockSpec((_BM, ncol), lambda i: (i, 0)), _full(y)]
    operands = [adj_bf, y]
    if mixT is not None:
        in_specs.append(pl.BlockSpec((kin, _BM), lambda i: (0, i)))
        operands.append(mixT)
    if w_next is not None:
        in_specs.append(_full(w_next))
        operands.append(w_next)
    out_specs, out_shape = [], []
    if want_t32:
        out_specs.append(row(kin))
        out_shape.append(jax.ShapeDtypeStruct((n, kin), _f32))
    if want_u32:
        out_specs.append(row(kin))
        out_shape.append(jax.ShapeDtypeStruct((n, kin), _f32))
    if want_ubf:
        out_specs.append(row(kin))
        out_shape.append(jax.ShapeDtypeStruct((n, kin), _bf16))
    if w_next is not None:
        kout = w_next.shape[1]
        out_specs.append(row(kout))
        out_shape.append(jax.ShapeDtypeStruct((n, kout), _f32))
    flags = (act, mixT is not None, w_next is not None,
             want_t32, want_u32, want_ubf)
    return pl.pallas_call(
        functools.partial(_spmmT_body, flags),
        grid=grid,
        in_specs=in_specs,
        out_specs=out_specs,
        out_shape=out_shape,
    )(*operands)


def _sim_body(zt_ref, zh_ref, out_ref, tbuf, tbufT, sems):
    i = pl.program_id(0)
    j = pl.program_id(1)
    g = pl.num_programs(1)

    @pl.when(j >= i)
    def _active():
        a = i * g - (i * (i - 1)) // 2 + (j - i)
        p = jax.lax.rem(a, 2)
        dst = out_ref.at[pl.ds(i * _BS, _BS), pl.ds(j * _BS, _BS)]
        dstT = out_ref.at[pl.ds(j * _BS, _BS), pl.ds(i * _BS, _BS)]
        cp = pltpu.make_async_copy(tbuf.at[p], dst, sems.at[p, 0])
        cpT = pltpu.make_async_copy(tbufT.at[p], dstT, sems.at[p, 1])

        @pl.when(a >= 2)
        def _drain_prev():
            cp.wait()
            cpT.wait()

        dims = (((1,), (1,)), ((), ()))
        zt_i = zt_ref[pl.ds(i * _BS, _BS), :]
        zt_j = zt_ref[pl.ds(j * _BS, _BS), :]
        zh_i = zh_ref[pl.ds(i * _BS, _BS), :]
        zh_j = zh_ref[pl.ds(j * _BS, _BS), :]
        l1 = jax.lax.dot_general(zt_i, zt_j, dims, preferred_element_type=_f32)
        l2 = jax.lax.dot_general(zh_i, zh_j, dims, preferred_element_type=_f32)
        tile = 1.0 + 0.5 * (jnp.tanh(0.5 * l1) + jnp.tanh(0.5 * l2))
        tbuf[p] = tile
        tbufT[p] = tile.T
        cp.start()
        cpT.start()

        @pl.when(jnp.logical_and(i == g - 1, j == g - 1))
        def _drain_all():
            cp.wait()
            cpT.wait()
            q = 1 - p
            pltpu.make_async_copy(tbuf.at[q], dst, sems.at[q, 0]).wait()
            pltpu.make_async_copy(tbufT.at[q], dstT, sems.at[q, 1]).wait()


def _similarity(ztl, zt_full, zhl, zh_full):
    del ztl, zhl
    n = zt_full.shape[0]
    g = n // _BS
    return pl.pallas_call(
        _sim_body,
        grid=(g, g),
        in_specs=[_full2(zt_full), _full2(zh_full)],
        out_specs=pl.BlockSpec(memory_space=pl.ANY),
        out_shape=jax.ShapeDtypeStruct((n, n), _f32),
        scratch_shapes=[
            pltpu.VMEM((2, _BS, _BS), _f32),
            pltpu.VMEM((2, _BS, _BS), _f32),
            pltpu.SemaphoreType.DMA((2, 2)),
        ],
    )(zt_full, zh_full)


def _full2(a):
    return pl.BlockSpec(a.shape, lambda i, j: (0, 0))


def _pipeline(X_tilde1, adj1, W_enc1, b_enc1, W_enc2, b_enc2, W_enc3, b_enc3,
              W_dec1, b_dec1, W_dec2, b_dec2, W_dec3, b_dec3, W_xbar, b_xbar,
              Wg1, Wg2, Wg3, Wg4, Wg5, Wg6):
    z_ae1, z_ae2T, z_ae3, z_ae3T, y1, x_hat = _ae_chain(
        X_tilde1, W_enc1, b_enc1, W_enc2, b_enc2, W_enc3, b_enc3,
        W_dec1, b_dec1, W_dec2, b_dec2, W_dec3, b_dec3,
        W_xbar, b_xbar, Wg1)

    y2, adj_bf = _spmm_stage(adj1, y1, mix=z_ae1, w_next=Wg2,
                             act=True, cast_out=True)
    (y3,) = _spmmT_stage(adj_bf, y2, mixT=z_ae2T, w_next=Wg3, act=True)
    z_igae3, z_tilde, zt_bf, y4 = _spmmT_stage(
        adj_bf, y3, mixT=z_ae3T, w_next=Wg4, act=False,
        want_t32=True, want_u32=True, want_ubf=True)
    (y5,) = _spmmT_stage(adj_bf, y4, w_next=Wg5, act=True)
    (y6,) = _spmm_stage(adj_bf, y5, w_next=Wg6, act=True)
    z_hat, zh_bf = _spmm_stage(adj_bf, y6, act=True,
                               want_t32=True, want_tbf=True)

    adj_hat = _similarity(zt_bf, zt_bf, zh_bf, zh_bf)
    return (x_hat, z_hat, adj_hat, z_ae3, z_igae3, z_tilde)


def kernel(X_tilde1, adj1, W_enc1, b_enc1, W_enc2, b_enc2, W_enc3, b_enc3,
           W_dec1, b_dec1, W_dec2, b_dec2, W_dec3, b_dec3, W_xbar, b_xbar,
           Wg1, Wg2, Wg3, Wg4, Wg5, Wg6):
    r = lambda b: b.reshape(1, -1)
    return _pipeline(X_tilde1, adj1, W_enc1, r(b_enc1), W_enc2, r(b_enc2),
                     W_enc3, r(b_enc3), W_dec1, r(b_dec1), W_dec2,
                     r(b_dec2), W_dec3, r(b_dec3), W_xbar, r(b_xbar),
                     Wg1, Wg2, Wg3, Wg4, Wg5, Wg6)


# speedup vs baseline: 1.0131x; 1.0131x over previous
import functools

import jax
import jax.numpy as jnp
from jax.experimental import pallas as pl
from jax.experimental.pallas import tpu as pltpu

_N = 4096
_SIGMA = 0.5
_BM = 512
_BS = 1024

_f32 = jnp.float32
_bf16 = jnp.bfloat16


def _dot(a, b):
    return jnp.dot(a.astype(_bf16), b.astype(_bf16),
                   preferred_element_type=_f32)


def _k1_body(x_ref, we1, be1, we2, be2, we3, be3,
             wd1, bd1, wd2, bd2, wd3, bd3, wxb, bxb, wg1,
             z1_ref, z2_ref, z3_ref, y1_ref, xhat_ref):
    x = x_ref[...]
    z1 = jax.nn.relu(_dot(x, we1[...]) + be1[...])
    z2 = jax.nn.relu(_dot(z1, we2[...]) + be2[...])
    z3 = _dot(z2, we3[...]) + be3[...]
    z1_ref[...] = z1
    z2_ref[...] = z2
    z3_ref[...] = z3
    y1_ref[...] = _dot(x, wg1[...])
    d = jax.nn.relu(_dot(z3, wd1[...]) + bd1[...])
    d = jax.nn.relu(_dot(d, wd2[...]) + bd2[...])
    d = jax.nn.relu(_dot(d, wd3[...]) + bd3[...])
    xhat_ref[...] = _dot(d, wxb[...]) + bxb[...]


def _full(a):
    nd = a.ndim
    return pl.BlockSpec(a.shape, lambda i: (0,) * nd)


def _ae_chain(x, we1, be1, we2, be2, we3, be3,
              wd1, bd1, wd2, bd2, wd3, bd3, wxb, bxb, wg1_bf):
    n = x.shape[0]
    grid = (n // _BM,)
    row = lambda k: pl.BlockSpec((_BM, k), lambda i: (i, 0))
    consts = [we1, be1, we2, be2, we3, be3,
              wd1, bd1, wd2, bd2, wd3, bd3, wxb, bxb, wg1_bf]
    return pl.pallas_call(
        _k1_body,
        grid=grid,
        in_specs=[row(512)] + [_full(c) for c in consts],
        out_specs=[row(128), row(64), row(20), row(128), row(512)],
        out_shape=[
            jax.ShapeDtypeStruct((n, 128), _f32),
            jax.ShapeDtypeStruct((n, 64), _f32),
            jax.ShapeDtypeStruct((n, 20), _f32),
            jax.ShapeDtypeStruct((n, 128), _f32),
            jax.ShapeDtypeStruct((n, 512), _f32),
        ],
    )(x, *consts)


def _spmm_body(flags, *refs):
    (act, has_mix, has_next, want_t32, want_tbf, want_u32, want_ubf,
     cast_out) = flags
    it = iter(refs)
    adj_ref = next(it)
    y_ref = next(it)
    mix_ref = next(it) if has_mix else None
    w_ref = next(it) if has_next else None
    adj_blk = adj_ref[...].astype(_bf16) if cast_out else adj_ref[...]
    acc = jnp.dot(adj_blk, y_ref[...].astype(_bf16),
                  preferred_element_type=_f32)
    t = jnp.tanh(acc) if act else acc
    u = (1.0 - _SIGMA) * mix_ref[...] + _SIGMA * t if has_mix else t
    if want_t32:
        next(it)[...] = t
    if want_tbf:
        next(it)[...] = t.astype(_bf16)
    if want_u32:
        next(it)[...] = u
    if want_ubf:
        next(it)[...] = u.astype(_bf16)
    if has_next:
        next(it)[...] = _dot(u, w_ref[...])
    if cast_out:
        next(it)[...] = adj_blk


def _spmm_stage(adj_bf, y, mix=None, w_next=None, act=True,
                want_t32=False, want_tbf=False, want_u32=False,
                want_ubf=False, cast_out=False):
    n = adj_bf.shape[0]
    ncol = adj_bf.shape[1]
    kin = y.shape[1]
    grid = (n // _BM,)
    row = lambda k: pl.BlockSpec((_BM, k), lambda i: (i, 0))
    in_specs = [pl.BlockSpec((_BM, ncol), lambda i: (i, 0)), _full(y)]
    operands = [adj_bf, y]
    if mix is not None:
        in_specs.append(row(kin))
        operands.append(mix)
    if w_next is not None:
        in_specs.append(_full(w_next))
        operands.append(w_next)
    out_specs, out_shape = [], []
    if want_t32:
        out_specs.append(row(kin))
        out_shape.append(jax.ShapeDtypeStruct((n, kin), _f32))
    if want_tbf:
        out_specs.append(row(kin))
        out_shape.append(jax.ShapeDtypeStruct((n, kin), _bf16))
    if want_u32:
        out_specs.append(row(kin))
        out_shape.append(jax.ShapeDtypeStruct((n, kin), _f32))
    if want_ubf:
        out_specs.append(row(kin))
        out_shape.append(jax.ShapeDtypeStruct((n, kin), _bf16))
    if w_next is not None:
        kout = w_next.shape[1]
        out_specs.append(row(kout))
        out_shape.append(jax.ShapeDtypeStruct((n, kout), _f32))
    if cast_out:
        out_specs.append(pl.BlockSpec((_BM, ncol), lambda i: (i, 0)))
        out_shape.append(jax.ShapeDtypeStruct((n, ncol), _bf16))
    flags = (act, mix is not None, w_next is not None,
             want_t32, want_tbf, want_u32, want_ubf, cast_out)
    outs = pl.pallas_call(
        functools.partial(_spmm_body, flags),
        grid=grid,
        in_specs=in_specs,
        out_specs=out_specs,
        out_shape=out_shape,
    )(*operands)
    return outs


def _sim_body(zt_ref, zh_ref, out_ref, tbuf, tbufT, sems):
    i = pl.program_id(0)
    j = pl.program_id(1)
    g = pl.num_programs(1)

    @pl.when(j >= i)
    def _active():
        a = i * g - (i * (i - 1)) // 2 + (j - i)
        p = jax.lax.rem(a, 2)
        dst = out_ref.at[pl.ds(i * _BS, _BS), pl.ds(j * _BS, _BS)]
        dstT = out_ref.at[pl.ds(j * _BS, _BS), pl.ds(i * _BS, _BS)]
        cp = pltpu.make_async_copy(tbuf.at[p], dst, sems.at[p, 0])
        cpT = pltpu.make_async_copy(tbufT.at[p], dstT, sems.at[p, 1])

        @pl.when(a >= 2)
        def _drain_prev():
            cp.wait()
            cpT.wait()

        dims = (((1,), (1,)), ((), ()))
        zt_i = zt_ref[pl.ds(i * _BS, _BS), :]
        zt_j = zt_ref[pl.ds(j * _BS, _BS), :]
        zh_i = zh_ref[pl.ds(i * _BS, _BS), :]
        zh_j = zh_ref[pl.ds(j * _BS, _BS), :]
        l1 = jax.lax.dot_general(zt_i, zt_j, dims, preferred_element_type=_f32)
        l2 = jax.lax.dot_general(zh_i, zh_j, dims, preferred_element_type=_f32)
        tile = 1.0 + 0.5 * (jnp.tanh(0.5 * l1) + jnp.tanh(0.5 * l2))
        tbuf[p] = tile
        tbufT[p] = tile.T
        cp.start()
        cpT.start()

        @pl.when(jnp.logical_and(i == g - 1, j == g - 1))
        def _drain_all():
            cp.wait()
            cpT.wait()
            q = 1 - p
            pltpu.make_async_copy(tbuf.at[q], dst, sems.at[q, 0]).wait()
            pltpu.make_async_copy(tbufT.at[q], dstT, sems.at[q, 1]).wait()


def _similarity(ztl, zt_full, zhl, zh_full):
    del ztl, zhl
    n = zt_full.shape[0]
    g = n // _BS
    return pl.pallas_call(
        _sim_body,
        grid=(g, g),
        in_specs=[_full2(zt_full), _full2(zh_full)],
        out_specs=pl.BlockSpec(memory_space=pl.ANY),
        out_shape=jax.ShapeDtypeStruct((n, n), _f32),
        scratch_shapes=[
            pltpu.VMEM((2, _BS, _BS), _f32),
            pltpu.VMEM((2, _BS, _BS), _f32),
            pltpu.SemaphoreType.DMA((2, 2)),
        ],
    )(zt_full, zh_full)


def _full2(a):
    return pl.BlockSpec(a.shape, lambda i, j: (0, 0))


def _pipeline(X_tilde1, adj1, W_enc1, b_enc1, W_enc2, b_enc2, W_enc3, b_enc3,
              W_dec1, b_dec1, W_dec2, b_dec2, W_dec3, b_dec3, W_xbar, b_xbar,
              Wg1, Wg2, Wg3, Wg4, Wg5, Wg6):
    z_ae1, z_ae2, z_ae3, y1, x_hat = _ae_chain(
        X_tilde1, W_enc1, b_enc1, W_enc2, b_enc2, W_enc3, b_enc3,
        W_dec1, b_dec1, W_dec2, b_dec2, W_dec3, b_dec3,
        W_xbar, b_xbar, Wg1)

    y2, adj_bf = _spmm_stage(adj1, y1, mix=z_ae1, w_next=Wg2,
                             act=True, cast_out=True)
    (y3,) = _spmm_stage(adj_bf, y2, mix=z_ae2, w_next=Wg3, act=True)
    z_igae3, z_tilde, zt_bf, y4 = _spmm_stage(
        adj_bf, y3, mix=z_ae3, w_next=Wg4, act=False,
        want_t32=True, want_u32=True, want_ubf=True)
    (y5,) = _spmm_stage(adj_bf, y4, w_next=Wg5, act=True)
    (y6,) = _spmm_stage(adj_bf, y5, w_next=Wg6, act=True)
    z_hat, zh_bf = _spmm_stage(adj_bf, y6, act=True,
                               want_t32=True, want_tbf=True)

    adj_hat = _similarity(zt_bf, zt_bf, zh_bf, zh_bf)
    return (x_hat, z_hat, adj_hat, z_ae3, z_igae3, z_tilde)


def kernel(X_tilde1, adj1, W_enc1, b_enc1, W_enc2, b_enc2, W_enc3, b_enc3,
           W_dec1, b_dec1, W_dec2, b_dec2, W_dec3, b_dec3, W_xbar, b_xbar,
           Wg1, Wg2, Wg3, Wg4, Wg5, Wg6):
    r = lambda b: b.reshape(1, -1)
    return _pipeline(X_tilde1, adj1, W_enc1, r(b_enc1), W_enc2, r(b_enc2),
                     W_enc3, r(b_enc3), W_dec1, r(b_dec1), W_dec2,
                     r(b_dec2), W_dec3, r(b_dec3), W_xbar, r(b_xbar),
                     Wg1, Wg2, Wg3, Wg4, Wg5, Wg6)


# speedup vs baseline: 1.0361x; 1.0227x over previous
import functools

import jax
import jax.numpy as jnp
from jax.experimental import pallas as pl
from jax.experimental.pallas import tpu as pltpu

_N = 4096
_SIGMA = 0.5
_BM = 1024
_BS = 1024

_f32 = jnp.float32
_bf16 = jnp.bfloat16


def _dot(a, b):
    return jnp.dot(a.astype(_bf16), b.astype(_bf16),
                   preferred_element_type=_f32)


def _k1_body(x_ref, we1, be1, we2, be2, we3, be3,
             wd1, bd1, wd2, bd2, wd3, bd3, wxb, bxb, wg1,
             z1_ref, z2_ref, z3_ref, y1_ref, xhat_ref):
    x = x_ref[...]
    z1 = jax.nn.relu(_dot(x, we1[...]) + be1[...])
    z2 = jax.nn.relu(_dot(z1, we2[...]) + be2[...])
    z3 = _dot(z2, we3[...]) + be3[...]
    z1_ref[...] = z1
    z2_ref[...] = z2
    z3_ref[...] = z3
    y1_ref[...] = _dot(x, wg1[...])
    d = jax.nn.relu(_dot(z3, wd1[...]) + bd1[...])
    d = jax.nn.relu(_dot(d, wd2[...]) + bd2[...])
    d = jax.nn.relu(_dot(d, wd3[...]) + bd3[...])
    xhat_ref[...] = _dot(d, wxb[...]) + bxb[...]


def _full(a):
    nd = a.ndim
    return pl.BlockSpec(a.shape, lambda i: (0,) * nd)


def _ae_chain(x, we1, be1, we2, be2, we3, be3,
              wd1, bd1, wd2, bd2, wd3, bd3, wxb, bxb, wg1_bf):
    n = x.shape[0]
    grid = (n // _BM,)
    row = lambda k: pl.BlockSpec((_BM, k), lambda i: (i, 0))
    consts = [we1, be1, we2, be2, we3, be3,
              wd1, bd1, wd2, bd2, wd3, bd3, wxb, bxb, wg1_bf]
    return pl.pallas_call(
        _k1_body,
        grid=grid,
        in_specs=[row(512)] + [_full(c) for c in consts],
        out_specs=[row(128), row(64), row(20), row(128), row(512)],
        out_shape=[
            jax.ShapeDtypeStruct((n, 128), _f32),
            jax.ShapeDtypeStruct((n, 64), _f32),
            jax.ShapeDtypeStruct((n, 20), _f32),
            jax.ShapeDtypeStruct((n, 128), _f32),
            jax.ShapeDtypeStruct((n, 512), _f32),
        ],
    )(x, *consts)


def _spmm_body(flags, *refs):
    (act, has_mix, has_next, want_t32, want_tbf, want_u32, want_ubf,
     cast_out) = flags
    it = iter(refs)
    adj_ref = next(it)
    y_ref = next(it)
    mix_ref = next(it) if has_mix else None
    w_ref = next(it) if has_next else None
    adj_blk = adj_ref[...].astype(_bf16) if cast_out else adj_ref[...]
    acc = jnp.dot(adj_blk, y_ref[...].astype(_bf16),
                  preferred_element_type=_f32)
    t = jnp.tanh(acc) if act else acc
    u = (1.0 - _SIGMA) * mix_ref[...] + _SIGMA * t if has_mix else t
    if want_t32:
        next(it)[...] = t
    if want_tbf:
        next(it)[...] = t.astype(_bf16)
    if want_u32:
        next(it)[...] = u
    if want_ubf:
        next(it)[...] = u.astype(_bf16)
    if has_next:
        next(it)[...] = _dot(u, w_ref[...])
    if cast_out:
        next(it)[...] = adj_blk


def _spmm_stage(adj_bf, y, mix=None, w_next=None, act=True,
                want_t32=False, want_tbf=False, want_u32=False,
                want_ubf=False, cast_out=False):
    n = adj_bf.shape[0]
    ncol = adj_bf.shape[1]
    kin = y.shape[1]
    grid = (n // _BM,)
    row = lambda k: pl.BlockSpec((_BM, k), lambda i: (i, 0))
    in_specs = [pl.BlockSpec((_BM, ncol), lambda i: (i, 0)), _full(y)]
    operands = [adj_bf, y]
    if mix is not None:
        in_specs.append(row(kin))
        operands.append(mix)
    if w_next is not None:
        in_specs.append(_full(w_next))
        operands.append(w_next)
    out_specs, out_shape = [], []
    if want_t32:
        out_specs.append(row(kin))
        out_shape.append(jax.ShapeDtypeStruct((n, kin), _f32))
    if want_tbf:
        out_specs.append(row(kin))
        out_shape.append(jax.ShapeDtypeStruct((n, kin), _bf16))
    if want_u32:
        out_specs.append(row(kin))
        out_shape.append(jax.ShapeDtypeStruct((n, kin), _f32))
    if want_ubf:
        out_specs.append(row(kin))
        out_shape.append(jax.ShapeDtypeStruct((n, kin), _bf16))
    if w_next is not None:
        kout = w_next.shape[1]
        out_specs.append(row(kout))
        out_shape.append(jax.ShapeDtypeStruct((n, kout), _f32))
    if cast_out:
        out_specs.append(pl.BlockSpec((_BM, ncol), lambda i: (i, 0)))
        out_shape.append(jax.ShapeDtypeStruct((n, ncol), _bf16))
    flags = (act, mix is not None, w_next is not None,
             want_t32, want_tbf, want_u32, want_ubf, cast_out)
    outs = pl.pallas_call(
        functools.partial(_spmm_body, flags),
        grid=grid,
        in_specs=in_specs,
        out_specs=out_specs,
        out_shape=out_shape,
    )(*operands)
    return outs


def _sim_body(zt_ref, zh_ref, out_ref, tbuf, tbufT, sems):
    i = pl.program_id(0)
    j = pl.program_id(1)
    g = pl.num_programs(1)

    @pl.when(j >= i)
    def _active():
        a = i * g - (i * (i - 1)) // 2 + (j - i)
        p = jax.lax.rem(a, 2)
        dst = out_ref.at[pl.ds(i * _BS, _BS), pl.ds(j * _BS, _BS)]
        dstT = out_ref.at[pl.ds(j * _BS, _BS), pl.ds(i * _BS, _BS)]
        cp = pltpu.make_async_copy(tbuf.at[p], dst, sems.at[p, 0])
        cpT = pltpu.make_async_copy(tbufT.at[p], dstT, sems.at[p, 1])

        @pl.when(a >= 2)
        def _drain_prev():
            cp.wait()
            cpT.wait()

        dims = (((1,), (1,)), ((), ()))
        zt_i = zt_ref[pl.ds(i * _BS, _BS), :]
        zt_j = zt_ref[pl.ds(j * _BS, _BS), :]
        zh_i = zh_ref[pl.ds(i * _BS, _BS), :]
        zh_j = zh_ref[pl.ds(j * _BS, _BS), :]
        l1 = jax.lax.dot_general(zt_i, zt_j, dims, preferred_element_type=_f32)
        l2 = jax.lax.dot_general(zh_i, zh_j, dims, preferred_element_type=_f32)
        tile = 1.0 + 0.5 * (jnp.tanh(0.5 * l1) + jnp.tanh(0.5 * l2))
        tbuf[p] = tile
        tbufT[p] = tile.T
        cp.start()
        cpT.start()

        @pl.when(jnp.logical_and(i == g - 1, j == g - 1))
        def _drain_all():
            cp.wait()
            cpT.wait()
            q = 1 - p
            pltpu.make_async_copy(tbuf.at[q], dst, sems.at[q, 0]).wait()
            pltpu.make_async_copy(tbufT.at[q], dstT, sems.at[q, 1]).wait()


def _similarity(ztl, zt_full, zhl, zh_full):
    del ztl, zhl
    n = zt_full.shape[0]
    g = n // _BS
    return pl.pallas_call(
        _sim_body,
        grid=(g, g),
        in_specs=[_full2(zt_full), _full2(zh_full)],
        out_specs=pl.BlockSpec(memory_space=pl.ANY),
        out_shape=jax.ShapeDtypeStruct((n, n), _f32),
        scratch_shapes=[
            pltpu.VMEM((2, _BS, _BS), _f32),
            pltpu.VMEM((2, _BS, _BS), _f32),
            pltpu.SemaphoreType.DMA((2, 2)),
        ],
    )(zt_full, zh_full)


def _full2(a):
    return pl.BlockSpec(a.shape, lambda i, j: (0, 0))


def _pipeline(X_tilde1, adj1, W_enc1, b_enc1, W_enc2, b_enc2, W_enc3, b_enc3,
              W_dec1, b_dec1, W_dec2, b_dec2, W_dec3, b_dec3, W_xbar, b_xbar,
              Wg1, Wg2, Wg3, Wg4, Wg5, Wg6):
    z_ae1, z_ae2, z_ae3, y1, x_hat = _ae_chain(
        X_tilde1, W_enc1, b_enc1, W_enc2, b_enc2, W_enc3, b_enc3,
        W_dec1, b_dec1, W_dec2, b_dec2, W_dec3, b_dec3,
        W_xbar, b_xbar, Wg1)

    y2, adj_bf = _spmm_stage(adj1, y1, mix=z_ae1, w_next=Wg2,
                             act=True, cast_out=True)
    (y3,) = _spmm_stage(adj_bf, y2, mix=z_ae2, w_next=Wg3, act=True)
    z_igae3, z_tilde, zt_bf, y4 = _spmm_stage(
        adj_bf, y3, mix=z_ae3, w_next=Wg4, act=False,
        want_t32=True, want_u32=True, want_ubf=True)
    (y5,) = _spmm_stage(adj_bf, y4, w_next=Wg5, act=True)
    (y6,) = _spmm_stage(adj_bf, y5, w_next=Wg6, act=True)
    z_hat, zh_bf = _spmm_stage(adj_bf, y6, act=True,
                               want_t32=True, want_tbf=True)

    adj_hat = _similarity(zt_bf, zt_bf, zh_bf, zh_bf)
    return (x_hat, z_hat, adj_hat, z_ae3, z_igae3, z_tilde)


def kernel(X_tilde1, adj1, W_enc1, b_enc1, W_enc2, b_enc2, W_enc3, b_enc3,
           W_dec1, b_dec1, W_dec2, b_dec2, W_dec3, b_dec3, W_xbar, b_xbar,
           Wg1, Wg2, Wg3, Wg4, Wg5, Wg6):
    r = lambda b: b.reshape(1, -1)
    return _pipeline(X_tilde1, adj1, W_enc1, r(b_enc1), W_enc2, r(b_enc2),
                     W_enc3, r(b_enc3), W_dec1, r(b_dec1), W_dec2,
                     r(b_dec2), W_dec3, r(b_dec3), W_xbar, r(b_xbar),
                     Wg1, Wg2, Wg3, Wg4, Wg5, Wg6)
